# Initial kernel scaffold; baseline (speedup 1.0000x reference)
import functools

import jax
import jax.numpy as jnp
from jax import lax
from jax.experimental import pallas as pl
from jax.experimental.pallas import tpu as pltpu
from jax.experimental.pallas import tpu_sc as plsc

L = 16


def _entity_pool_sc(table, idx, masks):
    n_rows, D = table.shape
    BE, K = idx.shape
    info = plsc.get_sparse_core_info()
    nw = info.num_cores * info.num_subcores
    epw = BE // nw

    mesh = plsc.VectorSubcoreMesh(core_axis_name="c", subcore_axis_name="s")

    @functools.partial(
        pl.kernel,
        mesh=mesh,
        out_type=jax.ShapeDtypeStruct((BE, D), jnp.float32),
        scratch_types=[
            pltpu.VMEM((epw, K), jnp.int32),
            pltpu.VMEM((epw, K), jnp.int32),
            pltpu.VMEM((K, D), jnp.float32),
            pltpu.VMEM((K, D), jnp.float32),
            pltpu.VMEM((epw, D), jnp.float32),
            pltpu.SemaphoreType.DMA,
            pltpu.SemaphoreType.DMA,
        ],
    )
    def run(table_hbm, idx_hbm, mask_hbm, out_hbm,
            idx_v, mask_v, buf0, buf1, out_v, sem0, sem1):
        wid = lax.axis_index("s") * info.num_cores + lax.axis_index("c")
        base = wid * epw
        pltpu.sync_copy(idx_hbm.at[pl.ds(base, epw), :], idx_v)
        pltpu.sync_copy(mask_hbm.at[pl.ds(base, epw), :], mask_v)

        bufs = (buf0, buf1)
        sems = (sem0, sem1)

        def start(e):
            pltpu.make_async_copy(
                table_hbm.at[idx_v.at[e]], bufs[e % 2], sems[e % 2]
            ).start()

        def wait(e):
            pltpu.make_async_copy(
                table_hbm.at[idx_v.at[e]], bufs[e % 2], sems[e % 2]
            ).wait()

        start(0)
        start(1)
        for e in range(epw):
            wait(e)
            buf = bufs[e % 2]
            splats = []
            for kk in range(K):
                m = mask_v[e, kk]
                bk = jnp.where(m == 0, jnp.float32(-1e30), jnp.float32(0.0))
                splats.append(jnp.full((L,), bk, dtype=jnp.float32))

            def cbody(c, carry, buf=buf, splats=splats, e=e):
                off = c * L
                acc = buf[0, pl.ds(off, L)] + splats[0]
                for kk in range(1, K):
                    acc = jnp.maximum(acc, buf[kk, pl.ds(off, L)] + splats[kk])
                out_v[e, pl.ds(off, L)] = acc
                return carry

            lax.fori_loop(0, D // L, cbody, 0)
            if e + 2 < epw:
                start(e + 2)
        pltpu.sync_copy(out_v, out_hbm.at[pl.ds(base, epw), :])

    return run(table, idx, masks)


def kernel(mention_reprs, entities, entity_masks):
    B, M, D = mention_reprs.shape
    _, E, K = entities.shape
    table = mention_reprs.reshape(B * M, D)
    idx = (entities + (jnp.arange(B, dtype=jnp.int32) * M)[:, None, None]
           ).reshape(B * E, K)
    masks = entity_masks.reshape(B * E, K)
    out = _entity_pool_sc(table, idx, masks)
    return out.reshape(B, E, D)


# speedup vs baseline: 8.7583x; 8.7583x over previous
import functools

import jax
import jax.numpy as jnp
from jax import lax
from jax.experimental import pallas as pl
from jax.experimental.pallas import tpu as pltpu
from jax.experimental.pallas import tpu_sc as plsc

L = 16


def _entity_pool_sc(table, idx, masks):
    n_rows, D = table.shape
    BE, K = idx.shape
    info = plsc.get_sparse_core_info()
    nw = info.num_cores * info.num_subcores
    epw = BE // nw

    mesh = plsc.VectorSubcoreMesh(core_axis_name="c", subcore_axis_name="s")

    @functools.partial(
        pl.kernel,
        mesh=mesh,
        out_type=jax.ShapeDtypeStruct((BE, D), jnp.float32),
        scratch_types=[
            pltpu.VMEM((epw, K), jnp.int32),
            pltpu.VMEM((epw, K), jnp.int32),
            pltpu.VMEM((K, D), jnp.float32),
            pltpu.VMEM((K, D), jnp.float32),
            pltpu.VMEM((epw, D), jnp.float32),
            pltpu.SemaphoreType.DMA,
            pltpu.SemaphoreType.DMA,
        ],
    )
    def run(table_hbm, idx_hbm, mask_hbm, out_hbm,
            idx_v, mask_v, buf0, buf1, out_v, sem0, sem1):
        wid = lax.axis_index("s") * info.num_cores + lax.axis_index("c")
        base = wid * epw
        pltpu.sync_copy(idx_hbm.at[pl.ds(base, epw), :], idx_v)
        pltpu.sync_copy(mask_hbm.at[pl.ds(base, epw), :], mask_v)

        bufs = (buf0, buf1)
        sems = (sem0, sem1)

        def start(e):
            pltpu.make_async_copy(
                table_hbm.at[idx_v.at[e]], bufs[e % 2], sems[e % 2]
            ).start()

        def wait(e):
            pltpu.make_async_copy(
                table_hbm.at[idx_v.at[e]], bufs[e % 2], sems[e % 2]
            ).wait()

        start(0)
        start(1)
        for e in range(epw):
            wait(e)
            buf = bufs[e % 2]
            splats = []
            for h in range(K // L):
                mv = mask_v[e, pl.ds(h * L, L)]
                bv = jnp.where(mv == 0, jnp.float32(-1e30), jnp.float32(0.0))
                for j in range(L):
                    splats.append(jnp.full((L,), bv[j], dtype=jnp.float32))

            def cbody(c, carry, buf=buf, splats=splats, e=e):
                off = c * L
                acc = buf[0, pl.ds(off, L)] + splats[0]
                for kk in range(1, K):
                    acc = jnp.maximum(acc, buf[kk, pl.ds(off, L)] + splats[kk])
                out_v[e, pl.ds(off, L)] = acc
                return carry

            lax.fori_loop(0, D // L, cbody, 0)
            if e + 2 < epw:
                start(e + 2)
        pltpu.sync_copy(out_v, out_hbm.at[pl.ds(base, epw), :])

    return run(table, idx, masks)


def kernel(mention_reprs, entities, entity_masks):
    B, M, D = mention_reprs.shape
    _, E, K = entities.shape
    table = mention_reprs.reshape(B * M, D)
    idx = (entities + (jnp.arange(B, dtype=jnp.int32) * M)[:, None, None]
           ).reshape(B * E, K)
    masks = entity_masks.reshape(B * E, K)
    out = _entity_pool_sc(table, idx, masks)
    return out.reshape(B, E, D)
